# baseline (speedup 1.0000x reference)
import functools

import jax
import jax.numpy as jnp
from jax import lax
from jax.experimental import pallas as pl
from jax.experimental.pallas import tpu as pltpu
from jax.experimental.pallas import tpu_sc as plsc

NBUF = 10
LOOKAHEAD = 6


@functools.lru_cache(maxsize=None)
def _make_gather(B, V, D):
    info = plsc.get_sparse_core_info()
    NC, NS = info.num_cores, info.num_subcores
    NW = NC * NS
    assert B % NW == 0
    b_per_w = B // NW
    CH = 64
    assert b_per_w % (NBUF * CH) == 0
    n_ch = b_per_w // CH
    n_grp = n_ch // NBUF
    mesh = plsc.VectorSubcoreMesh(core_axis_name="c", subcore_axis_name="s")

    @functools.partial(
        pl.kernel,
        mesh=mesh,
        out_type=jax.ShapeDtypeStruct((B, D), jnp.float32),
        scratch_types=[
            pltpu.VMEM((b_per_w,), jnp.int32),
            pltpu.VMEM((NBUF, CH, D), jnp.float32),
            pltpu.VMEM_SHARED((V, D), jnp.float32),
            [pltpu.SemaphoreType.DMA] * NBUF,
            [pltpu.SemaphoreType.DMA] * NBUF,
        ],
    )
    def k(idx_hbm, table_hbm, out_hbm, idx_v, rows_v, table_sp, gsem, wsem):
        wid = lax.axis_index("s") * NC + lax.axis_index("c")
        base = wid * b_per_w

        @pl.when(lax.axis_index("s") == 0)
        def _():
            pltpu.sync_copy(table_hbm, table_sp)

        pltpu.sync_copy(idx_hbm.at[pl.ds(base, b_per_w)], idx_v)
        plsc.subcore_barrier()

        def gather(c, j):
            return pltpu.make_async_copy(
                table_sp.at[idx_v.at[pl.ds(c * CH, CH)]], rows_v.at[j], gsem[j]
            )

        def write(c, j):
            return pltpu.make_async_copy(
                rows_v.at[j], out_hbm.at[pl.ds(base + c * CH, CH)], wsem[j]
            )

        for j in range(LOOKAHEAD):
            gather(j, j).start()

        def body(g, carry):
            c0 = NBUF * g
            for j in range(NBUF):
                c = c0 + j
                gather(c, j).wait()
                write(c, j).start()
                nxt = (j + LOOKAHEAD) % NBUF

                @pl.when(c + LOOKAHEAD < n_ch)
                def _(c=c, nxt=nxt):
                    @pl.when(c >= NBUF - LOOKAHEAD)
                    def _():
                        write(c + LOOKAHEAD - NBUF, nxt).wait()

                    gather(c + LOOKAHEAD, nxt).start()

            return carry

        lax.fori_loop(0, n_grp, body, 0)
        for j in range(NBUF):
            write(n_ch - NBUF + j, j).wait()

    return k


def kernel(x, table):
    b, t = x.shape
    flat = x.reshape(b * t).astype(jnp.int32)
    out = _make_gather(b * t, table.shape[0], table.shape[1])(flat, table)
    return out.reshape(b, t, table.shape[1])


# speedup vs baseline: 1.0042x; 1.0042x over previous
import functools

import jax
import jax.numpy as jnp
from jax import lax
from jax.experimental import pallas as pl
from jax.experimental.pallas import tpu as pltpu
from jax.experimental.pallas import tpu_sc as plsc

NBUF = 10
LOOKAHEAD = 6


@functools.lru_cache(maxsize=None)
def _make_gather(B, V, D):
    info = plsc.get_sparse_core_info()
    NC, NS = info.num_cores, info.num_subcores
    NW = NC * NS
    assert B % NW == 0
    b_per_w = B // NW
    CH = 64
    assert b_per_w % (NBUF * CH) == 0
    n_ch = b_per_w // CH
    n_grp = n_ch // NBUF
    mesh = plsc.VectorSubcoreMesh(core_axis_name="c", subcore_axis_name="s")

    @functools.partial(
        pl.kernel,
        mesh=mesh,
        out_type=jax.ShapeDtypeStruct((B, D), jnp.float32),
        scratch_types=[
            pltpu.VMEM((b_per_w,), jnp.int32),
            pltpu.VMEM((NBUF, CH, D), jnp.float32),
            pltpu.VMEM_SHARED((V, D), jnp.float32),
            [pltpu.SemaphoreType.DMA] * NBUF,
            [pltpu.SemaphoreType.DMA] * NBUF,
            pltpu.SemaphoreType.DMA,
        ],
    )
    def k(idx_hbm, table_hbm, out_hbm, idx_v, rows_v, table_sp, gsem, wsem,
          tsem):
        s = lax.axis_index("s")
        wid = s * NC + lax.axis_index("c")
        base = wid * b_per_w

        full, rem = V // NS + (1 if V % NS else 0), 0
        full = ((V + NS - 1) // NS + 7) // 8 * 8
        rem = V - (NS - 1) * full
        assert 0 < rem <= full and (NS - 1) * full % 8 == 0

        @pl.when(s < NS - 1)
        def _():
            pltpu.make_async_copy(
                table_hbm.at[pl.ds(s * full, full)],
                table_sp.at[pl.ds(s * full, full)], tsem,
            ).start()

        @pl.when(s == NS - 1)
        def _():
            pltpu.make_async_copy(
                table_hbm.at[pl.ds((NS - 1) * full, rem)],
                table_sp.at[pl.ds((NS - 1) * full, rem)], tsem,
            ).start()

        pltpu.sync_copy(idx_hbm.at[pl.ds(base, b_per_w)], idx_v)

        @pl.when(s < NS - 1)
        def _():
            pltpu.make_async_copy(
                table_hbm.at[pl.ds(0, full)], table_sp.at[pl.ds(0, full)],
                tsem,
            ).wait()

        @pl.when(s == NS - 1)
        def _():
            pltpu.make_async_copy(
                table_hbm.at[pl.ds(0, rem)], table_sp.at[pl.ds(0, rem)], tsem,
            ).wait()

        plsc.subcore_barrier()

        def gather(c, j):
            return pltpu.make_async_copy(
                table_sp.at[idx_v.at[pl.ds(c * CH, CH)]], rows_v.at[j], gsem[j]
            )

        def write(c, j):
            return pltpu.make_async_copy(
                rows_v.at[j], out_hbm.at[pl.ds(base + c * CH, CH)], wsem[j]
            )

        for j in range(LOOKAHEAD):
            gather(j, j).start()

        def body(g, carry):
            c0 = NBUF * g
            for j in range(NBUF):
                c = c0 + j
                gather(c, j).wait()
                write(c, j).start()
                nxt = (j + LOOKAHEAD) % NBUF

                @pl.when(c + LOOKAHEAD < n_ch)
                def _(c=c, nxt=nxt):
                    @pl.when(c >= NBUF - LOOKAHEAD)
                    def _():
                        write(c + LOOKAHEAD - NBUF, nxt).wait()

                    gather(c + LOOKAHEAD, nxt).start()

            return carry

        lax.fori_loop(0, n_grp, body, 0)
        for j in range(NBUF):
            write(n_ch - NBUF + j, j).wait()

    return k


def kernel(x, table):
    b, t = x.shape
    flat = x.reshape(b * t).astype(jnp.int32)
    out = _make_gather(b * t, table.shape[0], table.shape[1])(flat, table)
    return out.reshape(b, t, table.shape[1])


# speedup vs baseline: 1.0048x; 1.0006x over previous
import functools

import jax
import jax.numpy as jnp
from jax import lax
from jax.experimental import pallas as pl
from jax.experimental.pallas import tpu as pltpu
from jax.experimental.pallas import tpu_sc as plsc

NBUF = 10
LOOKAHEAD = 6


@functools.lru_cache(maxsize=None)
def _make_gather(B, V, D):
    info = plsc.get_sparse_core_info()
    NC, NS = info.num_cores, info.num_subcores
    NW = NC * NS
    assert B % NW == 0
    b_per_w = B // NW
    CH = 64
    assert b_per_w % (NBUF * CH) == 0
    n_ch = b_per_w // CH
    n_grp = n_ch // NBUF
    mesh = plsc.VectorSubcoreMesh(core_axis_name="c", subcore_axis_name="s")

    @functools.partial(
        pl.kernel,
        mesh=mesh,
        out_type=jax.ShapeDtypeStruct((B, D), jnp.float32),
        scratch_types=[
            pltpu.VMEM((b_per_w,), jnp.int32),
            pltpu.VMEM((NBUF, CH, D), jnp.float32),
            pltpu.VMEM_SHARED((V, D), jnp.float32),
            [pltpu.SemaphoreType.DMA] * NBUF,
            [pltpu.SemaphoreType.DMA] * NBUF,
            pltpu.SemaphoreType.DMA,
        ],
    )
    def k(idx_hbm, table_hbm, out_hbm, idx_v, rows_v, table_sp, gsem, wsem,
          tsem):
        s = lax.axis_index("s")
        wid = s * NC + lax.axis_index("c")
        base = wid * b_per_w

        full = ((V + NS - 1) // NS + 7) // 8 * 8
        rem = V - (NS - 1) * full
        assert 0 < rem <= full and (NS - 1) * full % 8 == 0

        @pl.when(s < NS - 1)
        def _():
            pltpu.make_async_copy(
                table_hbm.at[pl.ds(s * full, full)],
                table_sp.at[pl.ds(s * full, full)], tsem,
            ).start()

        @pl.when(s == NS - 1)
        def _():
            pltpu.make_async_copy(
                table_hbm.at[pl.ds((NS - 1) * full, rem)],
                table_sp.at[pl.ds((NS - 1) * full, rem)], tsem,
            ).start()

        pltpu.sync_copy(idx_hbm.at[pl.ds(base, b_per_w)], idx_v)

        @pl.when(s < NS - 1)
        def _():
            pltpu.make_async_copy(
                table_hbm.at[pl.ds(0, full)], table_sp.at[pl.ds(0, full)],
                tsem,
            ).wait()

        @pl.when(s == NS - 1)
        def _():
            pltpu.make_async_copy(
                table_hbm.at[pl.ds(0, rem)], table_sp.at[pl.ds(0, rem)], tsem,
            ).wait()

        plsc.subcore_barrier()

        def gather(c, j):
            return pltpu.make_async_copy(
                table_sp.at[idx_v.at[pl.ds(c * CH, CH)]], rows_v.at[j], gsem[j]
            )

        def write(c, j):
            return pltpu.make_async_copy(
                rows_v.at[j], out_hbm.at[pl.ds(base + c * CH, CH)], wsem[j]
            )

        for j in range(LOOKAHEAD):
            gather(j, j).start()

        def body(g, carry):
            c0 = NBUF * g
            for j in range(NBUF):
                c = c0 + j
                gather(c, j).wait()
                write(c, j).start()
                nxt = (j + LOOKAHEAD) % NBUF

                @pl.when(c + LOOKAHEAD < n_ch)
                def _(c=c, nxt=nxt):
                    @pl.when(c >= NBUF - LOOKAHEAD)
                    def _():
                        write(c + LOOKAHEAD - NBUF, nxt).wait()

                    gather(c + LOOKAHEAD, nxt).start()

            return carry

        lax.fori_loop(0, n_grp, body, 0)
        for j in range(NBUF):
            write(n_ch - NBUF + j, j).wait()

    return k


def kernel(x, table):
    b, t = x.shape
    flat = x.reshape(b * t).astype(jnp.int32)
    out = _make_gather(b * t, table.shape[0], table.shape[1])(flat, table)
    return out.reshape(b, t, table.shape[1])
